# baseline (speedup 1.0000x reference)
import jax
import jax.numpy as jnp
from jax.experimental import pallas as pl
from jax.experimental.pallas import tpu as pltpu

_N = 5000
_FIN = 64
_FOUT = 128
_L = 30
_DMLP = 2 * _L * _L
_H = 1024

_W1_CHUNKS = 5
_W23_CHUNKS = 4
_W4_CHUNKS = 4
_OUT_CHUNKS = 5

_DN_T = (((0,), (0,)), ((), ()))


def _dot(a, b):
    return jnp.dot(a, b, preferred_element_type=jnp.float32)


def _chunk_copies(src, dst, nrows, nchunks, sem):
    rows = nrows // nchunks
    return [pltpu.make_async_copy(src.at[pl.ds(i * rows, rows), :],
                                  dst.at[pl.ds(i * rows, rows), :], sem)
            for i in range(nchunks)]


def _body(tri_h, qr_h, qi_h, xr_h, xi_h,
          w1_h, b1_h, w2_h, b2_h, w3_h, b3_h, w4_h, b4_h, cw_h, cb_h,
          or_h, oi_h,
          tri_v, qr_v, qi_v, xr_v, xi_v,
          w1_v, b1_v, w2_v, b2_v, w3_v, b3_v, w4_v, b4_v, cw_v, cb_v,
          or_v, oi_v,
          sem_tri, sem_qx, sem_w1, sem_w2, sem_w3, sem_w4, sem_small,
          sem_out):
    tri_cp = pltpu.make_async_copy(tri_h, tri_v, sem_tri)
    tri_cp.start()
    w1_cps = _chunk_copies(w1_h, w1_v, _DMLP, _W1_CHUNKS, sem_w1)
    w2_cps = _chunk_copies(w2_h, w2_v, _H, _W23_CHUNKS, sem_w2)
    w3_cps = _chunk_copies(w3_h, w3_v, _H, _W23_CHUNKS, sem_w3)
    w4_cps = _chunk_copies(w4_h, w4_v, _H, _W4_CHUNKS, sem_w4)
    for cp in w1_cps + w2_cps + w3_cps + w4_cps:
        cp.start()
    small_cps = [pltpu.make_async_copy(s, d, sem_small)
                 for s, d in ((b1_h, b1_v), (b2_h, b2_v), (b3_h, b3_v),
                              (b4_h, b4_v), (cw_h, cw_v), (cb_h, cb_v))]
    for cp in small_cps:
        cp.start()
    qx_cps = [pltpu.make_async_copy(s, d, sem_qx)
              for s, d in ((qr_h, qr_v), (qi_h, qi_v),
                           (xr_h, xr_v), (xi_h, xi_v))]
    for cp in qx_cps:
        cp.start()

    tri_cp.wait()
    tri = tri_v[...]
    t2 = _dot(tri, tri)
    tcat2d = jnp.concatenate([tri, t2], axis=1)
    tcat = jnp.concatenate([tcat2d[i:i + 1, :] for i in range(_L)], axis=1)

    for cp in small_cps:
        cp.wait()
    for cp in w1_cps:
        cp.wait()
    x = jnp.maximum(_dot(tcat, w1_v[...]) + b1_v[...], 0.0)
    for cp in w2_cps:
        cp.wait()
    x = jnp.maximum(_dot(x, w2_v[...]) + b2_v[...], 0.0)
    for cp in w3_cps:
        cp.wait()
    x = jnp.maximum(_dot(x, w3_v[...]) + b3_v[...], 0.0)
    for cp in w4_cps:
        cp.wait()
    y4 = _dot(x, w4_v[...]) + b4_v[...]

    for cp in qx_cps:
        cp.wait()
    qr = qr_v[...]
    qi = qi_v[...]
    xr = xr_v[...]
    xi = xi_v[...]
    ar = jax.lax.dot_general(qr, xr, _DN_T, preferred_element_type=jnp.float32)
    ai = jax.lax.dot_general(qr, xi, _DN_T, preferred_element_type=jnp.float32)
    br = jax.lax.dot_general(qi, xr, _DN_T, preferred_element_type=jnp.float32)
    bi = jax.lax.dot_general(qi, xi, _DN_T, preferred_element_type=jnp.float32)
    u = ar + bi
    v = br - ai
    cw0 = cw_v[0]
    cw1 = cw_v[1]
    uw0 = _dot(u, cw0)
    uw1 = _dot(u, cw1)
    vw0 = _dot(v, cw0)
    vw1 = _dot(v, cw1)

    dint = jnp.concatenate(
        [y4[0:1, i * 60:(i + 1) * 60] for i in range(_L)], axis=0)
    row = jax.lax.broadcasted_iota(jnp.int32, (2 * _L, _L), 0)
    col = jax.lax.broadcasted_iota(jnp.int32, (2 * _L, _L), 1)
    s0 = (row == 2 * col).astype(jnp.float32)
    s1 = (row == 2 * col + 1).astype(jnp.float32)
    d0 = _dot(dint, s0)
    d1 = _dot(dint, s1)

    def sym(d, m):
        return 0.5 * (_dot(d, m)
                      + jax.lax.dot_general(d, m, _DN_T,
                                            preferred_element_type=jnp.float32))

    pu = sym(d0, uw0) + sym(d1, uw1)
    pv = sym(d0, vw0) + sym(d1, vw1)
    cb = cb_v[...]

    rows = _N // _OUT_CHUNKS
    out_cps = []
    for i in range(_OUT_CHUNKS):
        sl = pl.ds(i * rows, rows)
        qr_c = qr_v[sl, :]
        qi_c = qi_v[sl, :]
        or_v[sl, :] = _dot(qr_c, pu) + _dot(qi_c, pv) + cb
        oi_v[sl, :] = _dot(qi_c, pu) - _dot(qr_c, pv) + cb
        for src, dst in ((or_v, or_h), (oi_v, oi_h)):
            cp = pltpu.make_async_copy(src.at[sl, :], dst.at[sl, :], sem_out)
            cp.start()
            out_cps.append(cp)
    for cp in out_cps:
        cp.wait()


def _f32(shape):
    return jax.ShapeDtypeStruct(shape, jnp.float32)


@jax.jit
def kernel(real, imag, Tri, Qreal, Qimag, W1, b1, W2, b2, W3, b3, W4, b4,
           conv_w, conv_b):
    hbm = pl.BlockSpec(memory_space=pltpu.MemorySpace.HBM)
    vm = pltpu.MemorySpace.VMEM
    f32 = jnp.float32
    out_r, out_i = pl.pallas_call(
        _body,
        in_specs=[hbm] * 15,
        out_specs=[hbm, hbm],
        out_shape=[_f32((_N, _FOUT)), _f32((_N, _FOUT))],
        scratch_shapes=(
            [vm((_L, _L), f32), vm((_N, _L), f32), vm((_N, _L), f32),
             vm((_N, _FIN), f32), vm((_N, _FIN), f32),
             vm((_DMLP, _H), f32), vm((1, _H), f32),
             vm((_H, _H), f32), vm((1, _H), f32),
             vm((_H, _H), f32), vm((1, _H), f32),
             vm((_H, _DMLP), f32), vm((1, _DMLP), f32),
             vm((2, _FIN, _FOUT), f32), vm((1, _FOUT), f32),
             vm((_N, _FOUT), f32), vm((_N, _FOUT), f32)]
            + [pltpu.SemaphoreType.DMA] * 8),
    )(Tri, Qreal, Qimag, real, imag,
      W1, b1.reshape(1, -1), W2, b2.reshape(1, -1),
      W3, b3.reshape(1, -1), W4, b4.reshape(1, -1),
      conv_w, conv_b)
    return out_r, out_i


# speedup vs baseline: 1.0097x; 1.0097x over previous
import jax
import jax.numpy as jnp
from jax.experimental import pallas as pl
from jax.experimental.pallas import tpu as pltpu

_N = 5000
_FIN = 64
_FOUT = 128
_L = 30
_DMLP = 2 * _L * _L
_H = 1024

_W1_CHUNKS = 5
_W23_CHUNKS = 4
_W4_CHUNKS = 4
_OUT_CHUNKS = 5

_DN_T = (((0,), (0,)), ((), ()))


def _dot(a, b):
    return jnp.dot(a, b, preferred_element_type=jnp.float32)


def _chunk_copies(src, dst, nrows, nchunks, sem):
    rows = nrows // nchunks
    return [pltpu.make_async_copy(src.at[pl.ds(i * rows, rows), :],
                                  dst.at[pl.ds(i * rows, rows), :], sem)
            for i in range(nchunks)]


def _body(tri_h, qr_h, qi_h, xr_h, xi_h,
          w1_h, b1_h, w2_h, b2_h, w3_h, b3_h, w4_h, b4_h, cw_h, cb_h,
          or_h, oi_h,
          tri_v, qr_v, qi_v, xr_v, xi_v,
          w1_v, b1_v, w2_v, b2_v, w3_v, b3_v, w4_v, b4_v, cw_v, cb_v,
          or_v, oi_v,
          sem_tri, sem_qx, sem_w1, sem_w2, sem_w3, sem_w4, sem_small,
          sem_out):
    tri_cp = pltpu.make_async_copy(tri_h, tri_v, sem_tri)
    tri_cp.start()
    small_cps = [pltpu.make_async_copy(s, d, sem_small)
                 for s, d in ((b1_h, b1_v), (b2_h, b2_v), (b3_h, b3_v),
                              (b4_h, b4_v), (cw_h, cw_v), (cb_h, cb_v))]
    for cp in small_cps:
        cp.start()
    w1_cps = _chunk_copies(w1_h, w1_v, _DMLP, _W1_CHUNKS, sem_w1)
    w2_cps = _chunk_copies(w2_h, w2_v, _H, _W23_CHUNKS, sem_w2)
    w3_cps = _chunk_copies(w3_h, w3_v, _H, _W23_CHUNKS, sem_w3)
    w4_cps = _chunk_copies(w4_h, w4_v, _H, _W4_CHUNKS, sem_w4)
    qx_cps = [pltpu.make_async_copy(s, d, sem_qx)
              for s, d in ((qr_h, qr_v), (qi_h, qi_v),
                           (xr_h, xr_v), (xi_h, xi_v))]
    for cp in w1_cps + w2_cps:
        cp.start()

    tri_cp.wait()
    tri = tri_v[...]
    t2 = _dot(tri, tri)
    tcat2d = jnp.concatenate([tri, t2], axis=1)
    tcat = jnp.concatenate([tcat2d[i:i + 1, :] for i in range(_L)], axis=1)

    for cp in small_cps:
        cp.wait()
    for cp in w1_cps:
        cp.wait()
    for cp in w3_cps:
        cp.start()
    x = jnp.maximum(_dot(tcat, w1_v[...]) + b1_v[...], 0.0)
    for cp in w2_cps:
        cp.wait()
    for cp in w4_cps:
        cp.start()
    x = jnp.maximum(_dot(x, w2_v[...]) + b2_v[...], 0.0)
    for cp in w3_cps:
        cp.wait()
    for cp in qx_cps:
        cp.start()
    x = jnp.maximum(_dot(x, w3_v[...]) + b3_v[...], 0.0)
    for cp in w4_cps:
        cp.wait()
    y4 = _dot(x, w4_v[...]) + b4_v[...]

    for cp in qx_cps:
        cp.wait()
    qr = qr_v[...]
    qi = qi_v[...]
    xr = xr_v[...]
    xi = xi_v[...]
    ar = jax.lax.dot_general(qr, xr, _DN_T, preferred_element_type=jnp.float32)
    ai = jax.lax.dot_general(qr, xi, _DN_T, preferred_element_type=jnp.float32)
    br = jax.lax.dot_general(qi, xr, _DN_T, preferred_element_type=jnp.float32)
    bi = jax.lax.dot_general(qi, xi, _DN_T, preferred_element_type=jnp.float32)
    u = ar + bi
    v = br - ai
    cw0 = cw_v[0]
    cw1 = cw_v[1]
    uw0 = _dot(u, cw0)
    uw1 = _dot(u, cw1)
    vw0 = _dot(v, cw0)
    vw1 = _dot(v, cw1)

    dint = jnp.concatenate(
        [y4[0:1, i * 60:(i + 1) * 60] for i in range(_L)], axis=0)
    row = jax.lax.broadcasted_iota(jnp.int32, (2 * _L, _L), 0)
    col = jax.lax.broadcasted_iota(jnp.int32, (2 * _L, _L), 1)
    s0 = (row == 2 * col).astype(jnp.float32)
    s1 = (row == 2 * col + 1).astype(jnp.float32)
    d0 = _dot(dint, s0)
    d1 = _dot(dint, s1)

    def sym(d, m):
        return 0.5 * (_dot(d, m)
                      + jax.lax.dot_general(d, m, _DN_T,
                                            preferred_element_type=jnp.float32))

    pu = sym(d0, uw0) + sym(d1, uw1)
    pv = sym(d0, vw0) + sym(d1, vw1)
    cb = cb_v[...]

    rows = _N // _OUT_CHUNKS
    out_cps = []
    for i in range(_OUT_CHUNKS):
        sl = pl.ds(i * rows, rows)
        qr_c = qr_v[sl, :]
        qi_c = qi_v[sl, :]
        or_v[sl, :] = _dot(qr_c, pu) + _dot(qi_c, pv) + cb
        oi_v[sl, :] = _dot(qi_c, pu) - _dot(qr_c, pv) + cb
        for src, dst in ((or_v, or_h), (oi_v, oi_h)):
            cp = pltpu.make_async_copy(src.at[sl, :], dst.at[sl, :], sem_out)
            cp.start()
            out_cps.append(cp)
    for cp in out_cps:
        cp.wait()


def _f32(shape):
    return jax.ShapeDtypeStruct(shape, jnp.float32)


@jax.jit
def kernel(real, imag, Tri, Qreal, Qimag, W1, b1, W2, b2, W3, b3, W4, b4,
           conv_w, conv_b):
    hbm = pl.BlockSpec(memory_space=pltpu.MemorySpace.HBM)
    vm = pltpu.MemorySpace.VMEM
    f32 = jnp.float32
    out_r, out_i = pl.pallas_call(
        _body,
        in_specs=[hbm] * 15,
        out_specs=[hbm, hbm],
        out_shape=[_f32((_N, _FOUT)), _f32((_N, _FOUT))],
        scratch_shapes=(
            [vm((_L, _L), f32), vm((_N, _L), f32), vm((_N, _L), f32),
             vm((_N, _FIN), f32), vm((_N, _FIN), f32),
             vm((_DMLP, _H), f32), vm((1, _H), f32),
             vm((_H, _H), f32), vm((1, _H), f32),
             vm((_H, _H), f32), vm((1, _H), f32),
             vm((_H, _DMLP), f32), vm((1, _DMLP), f32),
             vm((2, _FIN, _FOUT), f32), vm((1, _FOUT), f32),
             vm((_N, _FOUT), f32), vm((_N, _FOUT), f32)]
            + [pltpu.SemaphoreType.DMA] * 8),
    )(Tri, Qreal, Qimag, real, imag,
      W1, b1.reshape(1, -1), W2, b2.reshape(1, -1),
      W3, b3.reshape(1, -1), W4, b4.reshape(1, -1),
      conv_w, conv_b)
    return out_r, out_i
